# Initial kernel scaffold; baseline (speedup 1.0000x reference)
import jax
import jax.numpy as jnp
from jax.experimental import pallas as pl


def kernel(x):
    raise NotImplementedError("write your pallas kernel here")


# speedup vs baseline: 3.3290x; 3.3290x over previous
import jax
import jax.numpy as jnp
from jax.experimental import pallas as pl

_TOPK = 8
_EPS = 1e-10


def _topk_mask_kernel(x_ref, g_ref, out_ref):
    z = x_ref[...] + g_ref[...]
    rows, cols = z.shape
    col = jax.lax.broadcasted_iota(jnp.int32, (rows, cols), 1)
    out = jnp.zeros((rows, cols), jnp.float32)
    for _ in range(_TOPK):
        m = jnp.max(z, axis=1, keepdims=True)
        idx = jnp.min(jnp.where(z == m, col, cols), axis=1, keepdims=True)
        sel = col == idx
        out = jnp.where(sel, 1.0, out)
        z = jnp.where(sel, -jnp.inf, z)
    out_ref[...] = out


def _gumbel_const(shape, dtype):
    u = jax.random.uniform(jax.random.key(1), shape, dtype=dtype)
    return -jnp.log(_EPS - jnp.log(u + _EPS))


def kernel(x):
    g = _gumbel_const(x.shape, x.dtype)
    return pl.pallas_call(
        _topk_mask_kernel,
        out_shape=jax.ShapeDtypeStruct(x.shape, jnp.float32),
    )(x, g)
